# baseline (speedup 1.0000x reference)
import jax
import jax.numpy as jnp
from jax import lax
from jax.experimental import pallas as pl
from jax.experimental.pallas import tpu as pltpu


def _body(x_hbm, out_hbm, buf2, tbuf, stage2, insem, outsem):
    b = pl.program_id(0)
    nb = pl.num_programs(0)
    slot = lax.rem(b, 2)
    nslot = lax.rem(b + 1, 2)

    @pl.when(b == 0)
    def _():
        pltpu.make_async_copy(x_hbm.at[0], buf2.at[0], insem.at[0]).start()

    pltpu.make_async_copy(x_hbm.at[b], buf2.at[slot], insem.at[slot]).wait()

    @pl.when(b + 1 < nb)
    def _():
        pltpu.make_async_copy(
            x_hbm.at[b + 1], buf2.at[nslot], insem.at[nslot]
        ).start()

    tbuf[...] = jnp.transpose(
        buf2[slot].reshape(24, 72, 1440), (2, 0, 1)
    )

    for p in range(9):
        step = b * 9 + p
        ss = lax.rem(step, 4)
        base = p * 160
        even = tbuf[base:base + 80, :12, :].reshape(960, 72)
        odd = tbuf[base + 80:base + 160, :, :].reshape(1920, 72)

        @pl.when(step >= 4)
        def _():
            pltpu.make_async_copy(
                stage2.at[ss], out_hbm.at[b, pl.ds(0, 2880), :], outsem.at[ss]
            ).wait()

        stage2[ss, :960] = even
        stage2[ss, 960:] = odd
        pltpu.make_async_copy(
            stage2.at[ss],
            out_hbm.at[b, pl.ds(p * 2880, 2880), :],
            outsem.at[ss],
        ).start()

    @pl.when(b == nb - 1)
    def _():
        for k in range(4):
            pltpu.make_async_copy(
                stage2.at[k], out_hbm.at[b, pl.ds(0, 2880), :], outsem.at[k]
            ).wait()


def kernel(x):
    B = x.shape[0]
    x3 = x.reshape(B, 1728, 1440)
    out = pl.pallas_call(
        _body,
        grid=(B,),
        in_specs=[pl.BlockSpec(memory_space=pl.ANY)],
        out_specs=pl.BlockSpec(memory_space=pl.ANY),
        out_shape=jax.ShapeDtypeStruct((B, 25920, 72), jnp.float32),
        scratch_shapes=[
            pltpu.VMEM((2, 1728, 1440), jnp.float32),
            pltpu.VMEM((1440, 24, 72), jnp.float32),
            pltpu.VMEM((4, 2880, 72), jnp.float32),
            pltpu.SemaphoreType.DMA((2,)),
            pltpu.SemaphoreType.DMA((4,)),
        ],
        compiler_params=pltpu.CompilerParams(
            vmem_limit_bytes=62 * 1024 * 1024,
        ),
    )(x3)
    return out


# speedup vs baseline: 1.0091x; 1.0091x over previous
import jax
import jax.numpy as jnp
from jax import lax
from jax.experimental import pallas as pl
from jax.experimental.pallas import tpu as pltpu


def _body(x_hbm, out_hbm, buf2, tbuf, stage2, insem, outsem):
    b = pl.program_id(0)
    nb = pl.num_programs(0)
    slot = lax.rem(b, 2)
    nslot = lax.rem(b + 1, 2)

    @pl.when(b == 0)
    def _():
        pltpu.make_async_copy(x_hbm.at[0], buf2.at[0], insem.at[0]).start()

    pltpu.make_async_copy(x_hbm.at[b], buf2.at[slot], insem.at[slot]).wait()

    @pl.when(b + 1 < nb)
    def _():
        pltpu.make_async_copy(
            x_hbm.at[b + 1], buf2.at[nslot], insem.at[nslot]
        ).start()

    tbuf[...] = jnp.transpose(
        buf2[slot].reshape(24, 72, 1440), (2, 0, 1)
    )

    for p in range(9):
        step = b * 9 + p
        ss = lax.rem(step, 4)

        @pl.when(step >= 4)
        def _():
            pltpu.make_async_copy(
                stage2.at[ss], out_hbm.at[b, pl.ds(0, 2880), :], outsem.at[ss]
            ).wait()

        base = p * 160
        even = tbuf[base:base + 80, :12, :].reshape(960, 72)
        odd = tbuf[base + 80:base + 160, :, :].reshape(1920, 72)
        stage2[ss, :960] = even
        stage2[ss, 960:] = odd
        pltpu.make_async_copy(
            stage2.at[ss],
            out_hbm.at[b, pl.ds(p * 2880, 2880), :],
            outsem.at[ss],
        ).start()

    @pl.when(b == nb - 1)
    def _():
        for k in range(4):
            pltpu.make_async_copy(
                stage2.at[k], out_hbm.at[b, pl.ds(0, 2880), :], outsem.at[k]
            ).wait()


def kernel(x):
    B = x.shape[0]
    x3 = x.reshape(B, 1728, 1440)
    out = pl.pallas_call(
        _body,
        grid=(B,),
        in_specs=[pl.BlockSpec(memory_space=pl.ANY)],
        out_specs=pl.BlockSpec(memory_space=pl.ANY),
        out_shape=jax.ShapeDtypeStruct((B, 25920, 72), jnp.float32),
        scratch_shapes=[
            pltpu.VMEM((2, 1728, 1440), jnp.float32),
            pltpu.VMEM((1440, 24, 72), jnp.float32),
            pltpu.VMEM((4, 2880, 72), jnp.float32),
            pltpu.SemaphoreType.DMA((2,)),
            pltpu.SemaphoreType.DMA((4,)),
        ],
        compiler_params=pltpu.CompilerParams(
            vmem_limit_bytes=62 * 1024 * 1024,
        ),
    )(x3)
    return out
